# Initial kernel scaffold; baseline (speedup 1.0000x reference)
import jax
import jax.numpy as jnp
from jax.experimental import pallas as pl


def kernel(distance_matrix, emb_table, distance_weights):
    raise NotImplementedError("write your pallas kernel here")


# speedup vs baseline: 197.8862x; 197.8862x over previous
import functools

import jax
import jax.numpy as jnp
from jax import lax
from jax.experimental import pallas as pl
from jax.experimental.pallas import tpu as pltpu
from jax.experimental.pallas import tpu_sc as plsc

_D_MODEL = 64
_MAX_DISTANCE = 100.0
_DISTANCE_BINS = 20
_BIN_SIZE = _MAX_DISTANCE / _DISTANCE_BINS

_NC = 2
_NS = 16
_NW = _NC * _NS
_L = 16

_TOTAL = 16 * 256 * 256
_PER_W = _TOTAL // _NW

_mesh = plsc.VectorSubcoreMesh(core_axis_name="c", subcore_axis_name="s")


@functools.partial(
    pl.kernel,
    mesh=_mesh,
    out_type=jax.ShapeDtypeStruct((_TOTAL,), jnp.float32),
    scratch_types=[
        pltpu.VMEM((_PER_W,), jnp.float32),
        pltpu.VMEM((32,), jnp.float32),
    ],
    compiler_params=pltpu.CompilerParams(needs_layout_passes=False),
)
def _sc_bin_lookup(d_hbm, w_hbm, out_hbm, buf_v, w_v):
    wid = lax.axis_index("s") * _NC + lax.axis_index("c")
    base = wid * _PER_W
    pltpu.sync_copy(w_hbm, w_v)
    pltpu.sync_copy(d_hbm.at[pl.ds(base, _PER_W)], buf_v)

    bin_size = jnp.float32(_BIN_SIZE)

    def body(i, carry):
        off = i * _L
        d = buf_v[pl.ds(off, _L)]
        b = jnp.clip((d / bin_size).astype(jnp.int32), 0, _DISTANCE_BINS)
        buf_v[pl.ds(off, _L)] = plsc.load_gather(w_v, [b])
        return carry

    lax.fori_loop(0, _PER_W // _L, body, 0)
    pltpu.sync_copy(buf_v, out_hbm.at[pl.ds(base, _PER_W)])


def kernel(distance_matrix, emb_table, distance_weights):
    del emb_table
    d_flat = distance_matrix.reshape(_TOTAL)
    w_pad = jnp.zeros((32,), jnp.float32).at[: _DISTANCE_BINS + 1].set(
        distance_weights
    )
    out = _sc_bin_lookup(d_flat, w_pad)
    return out.reshape(16, 1, 256, 256)


# speedup vs baseline: 353.8266x; 1.7880x over previous
import functools

import jax
import jax.numpy as jnp
from jax import lax
from jax.experimental import pallas as pl
from jax.experimental.pallas import tpu as pltpu
from jax.experimental.pallas import tpu_sc as plsc

_D_MODEL = 64
_MAX_DISTANCE = 100.0
_DISTANCE_BINS = 20
_BIN_SIZE = _MAX_DISTANCE / _DISTANCE_BINS

_NC = 2
_NS = 16
_NW = _NC * _NS
_L = 16

_TOTAL = 16 * 256 * 256
_PER_W = _TOTAL // _NW

_mesh = plsc.VectorSubcoreMesh(core_axis_name="c", subcore_axis_name="s")


@functools.partial(
    pl.kernel,
    mesh=_mesh,
    out_type=jax.ShapeDtypeStruct((_TOTAL,), jnp.float32),
    scratch_types=[
        pltpu.VMEM((_PER_W,), jnp.float32),
        pltpu.VMEM((32,), jnp.float32),
    ],
    compiler_params=pltpu.CompilerParams(needs_layout_passes=False),
)
def _sc_bin_lookup(d_hbm, w_hbm, out_hbm, buf_v, w_v):
    wid = lax.axis_index("s") * _NC + lax.axis_index("c")
    base = wid * _PER_W
    pltpu.sync_copy(w_hbm, w_v)
    pltpu.sync_copy(d_hbm.at[pl.ds(base, _PER_W)], buf_v)

    bin_size = jnp.float32(_BIN_SIZE)

    @plsc.parallel_loop(0, _PER_W, _L, unroll=8)
    def _body(off):
        d = buf_v[pl.ds(off, _L)]
        b = jnp.clip((d / bin_size).astype(jnp.int32), 0, _DISTANCE_BINS)
        buf_v[pl.ds(off, _L)] = plsc.load_gather(w_v, [b])
    pltpu.sync_copy(buf_v, out_hbm.at[pl.ds(base, _PER_W)])


def kernel(distance_matrix, emb_table, distance_weights):
    del emb_table
    d_flat = distance_matrix.reshape(_TOTAL)
    w_pad = jnp.zeros((32,), jnp.float32).at[: _DISTANCE_BINS + 1].set(
        distance_weights
    )
    out = _sc_bin_lookup(d_flat, w_pad)
    return out.reshape(16, 1, 256, 256)


# speedup vs baseline: 421.4628x; 1.1912x over previous
import functools

import jax
import jax.numpy as jnp
from jax import lax
from jax.experimental import pallas as pl
from jax.experimental.pallas import tpu as pltpu
from jax.experimental.pallas import tpu_sc as plsc

_DISTANCE_BINS = 20
_BIN_SIZE = 100.0 / _DISTANCE_BINS

_NC = 2
_NS = 16
_L = 16

_ROWS = 128
_COLS = 256
_PER_W = _ROWS * _COLS

_mesh = plsc.VectorSubcoreMesh(core_axis_name="c", subcore_axis_name="s")


@functools.partial(
    pl.kernel,
    mesh=_mesh,
    out_type=jax.ShapeDtypeStruct((16, 1, 256, 256), jnp.float32),
    scratch_types=[
        pltpu.VMEM((_ROWS, _COLS), jnp.float32),
        pltpu.VMEM((_DISTANCE_BINS + 1,), jnp.float32),
    ],
    compiler_params=pltpu.CompilerParams(
        needs_layout_passes=False,
        use_tc_tiling_on_sc=True,
    ),
)
def _sc_bin_lookup(d_hbm, w_hbm, out_hbm, buf_v, w_v):
    wid = lax.axis_index("s") * _NC + lax.axis_index("c")
    b = wid // 2
    row0 = (wid % 2) * _ROWS
    pltpu.sync_copy(w_hbm, w_v)
    pltpu.sync_copy(d_hbm.at[b, pl.ds(row0, _ROWS)], buf_v)

    bin_size = jnp.float32(_BIN_SIZE)

    @plsc.parallel_loop(0, _ROWS, 1)
    def _row(r):
        @plsc.parallel_loop(0, _COLS, _L, unroll=8)
        def _vec(off):
            d = buf_v[r, pl.ds(off, _L)]
            bins = jnp.clip((d / bin_size).astype(jnp.int32), 0, _DISTANCE_BINS)
            buf_v[r, pl.ds(off, _L)] = plsc.load_gather(w_v, [bins])

    pltpu.sync_copy(buf_v, out_hbm.at[b, 0, pl.ds(row0, _ROWS)])


def kernel(distance_matrix, emb_table, distance_weights):
    del emb_table
    return _sc_bin_lookup(distance_matrix, distance_weights)


# speedup vs baseline: 474.1724x; 1.1251x over previous
import functools

import jax
import jax.numpy as jnp
from jax import lax
from jax.experimental import pallas as pl
from jax.experimental.pallas import tpu as pltpu
from jax.experimental.pallas import tpu_sc as plsc

_DISTANCE_BINS = 20
_BIN_SIZE = 100.0 / _DISTANCE_BINS

_NC = 2
_NS = 16
_L = 16

_ROWS = 128
_COLS = 256
_PER_W = _ROWS * _COLS

_mesh = plsc.VectorSubcoreMesh(core_axis_name="c", subcore_axis_name="s")


@functools.partial(
    pl.kernel,
    mesh=_mesh,
    out_type=jax.ShapeDtypeStruct((16, 1, 256, 256), jnp.float32),
    scratch_types=[
        pltpu.VMEM((_ROWS, _COLS), jnp.float32),
        pltpu.VMEM((_DISTANCE_BINS + 1,), jnp.float32),
    ],
    compiler_params=pltpu.CompilerParams(
        needs_layout_passes=False,
        use_tc_tiling_on_sc=True,
    ),
)
def _sc_bin_lookup(d_hbm, w_hbm, out_hbm, buf_v, w_v):
    wid = lax.axis_index("s") * _NC + lax.axis_index("c")
    b = wid // 2
    row0 = (wid % 2) * _ROWS
    pltpu.sync_copy(w_hbm, w_v)
    pltpu.sync_copy(d_hbm.at[b, pl.ds(row0, _ROWS)], buf_v)

    bin_size = jnp.float32(_BIN_SIZE)

    @plsc.parallel_loop(0, _ROWS, 1)
    def _row(r):
        for k in range(_COLS // _L):
            d = buf_v[r, pl.ds(k * _L, _L)]
            bins = jnp.clip((d / bin_size).astype(jnp.int32), 0, _DISTANCE_BINS)
            buf_v[r, pl.ds(k * _L, _L)] = plsc.load_gather(w_v, [bins])

    pltpu.sync_copy(buf_v, out_hbm.at[b, 0, pl.ds(row0, _ROWS)])


def kernel(distance_matrix, emb_table, distance_weights):
    del emb_table
    return _sc_bin_lookup(distance_matrix, distance_weights)


# speedup vs baseline: 475.2279x; 1.0022x over previous
import functools

import jax
import jax.numpy as jnp
from jax import lax
from jax.experimental import pallas as pl
from jax.experimental.pallas import tpu as pltpu
from jax.experimental.pallas import tpu_sc as plsc

_DISTANCE_BINS = 20
_BIN_SIZE = 100.0 / _DISTANCE_BINS

_NC = 2
_NS = 16
_L = 16

_ROWS = 128
_COLS = 256
_NCH = 4
_CH_ROWS = _ROWS // _NCH
_NBUF = 4

_mesh = plsc.VectorSubcoreMesh(core_axis_name="c", subcore_axis_name="s")


@functools.partial(
    pl.kernel,
    mesh=_mesh,
    out_type=jax.ShapeDtypeStruct((16, 1, 256, 256), jnp.float32),
    scratch_types=[
        pltpu.VMEM((_NBUF, _CH_ROWS, _COLS), jnp.float32),
        pltpu.VMEM((_DISTANCE_BINS + 1,), jnp.float32),
        pltpu.SemaphoreType.DMA,
        pltpu.SemaphoreType.DMA,
        pltpu.SemaphoreType.DMA,
        pltpu.SemaphoreType.DMA,
        pltpu.SemaphoreType.DMA,
        pltpu.SemaphoreType.DMA,
        pltpu.SemaphoreType.DMA,
        pltpu.SemaphoreType.DMA,
        pltpu.SemaphoreType.DMA,
    ],
    compiler_params=pltpu.CompilerParams(
        needs_layout_passes=False,
        use_tc_tiling_on_sc=True,
    ),
)
def _sc_bin_lookup(d_hbm, w_hbm, out_hbm, buf_v, w_v, w_sem, *sems):
    in_sems = sems[:_NBUF]
    out_sems = sems[_NBUF:]
    wid = lax.axis_index("s") * _NC + lax.axis_index("c")
    b = wid // 2
    row0 = (wid % 2) * _ROWS

    w_dma = pltpu.async_copy(w_hbm, w_v, w_sem)

    def start_in(c):
        return pltpu.async_copy(
            d_hbm.at[b, pl.ds(row0 + c * _CH_ROWS, _CH_ROWS)],
            buf_v.at[c % _NBUF],
            in_sems[c % _NBUF],
        )

    def start_out(c):
        return pltpu.async_copy(
            buf_v.at[c % _NBUF],
            out_hbm.at[b, 0, pl.ds(row0 + c * _CH_ROWS, _CH_ROWS)],
            out_sems[c % _NBUF],
        )

    bin_size = jnp.float32(_BIN_SIZE)

    def compute(c):
        bi = c % _NBUF

        @plsc.parallel_loop(0, _CH_ROWS, 1)
        def _row(r):
            for k in range(_COLS // _L):
                d = buf_v[bi, r, pl.ds(k * _L, _L)]
                bins = jnp.clip(
                    (d / bin_size).astype(jnp.int32), 0, _DISTANCE_BINS
                )
                buf_v[bi, r, pl.ds(k * _L, _L)] = plsc.load_gather(w_v, [bins])

    in_dmas = [start_in(0), start_in(1)]
    w_dma.wait()
    out_dmas = []
    for c in range(_NCH):
        in_dmas[c].wait()
        compute(c)
        out_dmas.append(start_out(c))
        if c + 2 < _NCH:
            in_dmas.append(start_in(c + 2))
    for dma in out_dmas:
        dma.wait()


def kernel(distance_matrix, emb_table, distance_weights):
    del emb_table
    return _sc_bin_lookup(distance_matrix, distance_weights)
